# Initial kernel scaffold; baseline (speedup 1.0000x reference)
import jax
import jax.numpy as jnp
from jax.experimental import pallas as pl


def kernel(indices, table):
    raise NotImplementedError("write your pallas kernel here")


# speedup vs baseline: 2.9761x; 2.9761x over previous
import functools

import jax
import jax.numpy as jnp
from jax import lax
from jax.experimental import pallas as pl
from jax.experimental.pallas import tpu as pltpu
from jax.experimental.pallas import tpu_sc as plsc

DIM = 128
NC = 2
NS = 16
NW = NC * NS
CHUNK = 128


def _emb_body(nchunk, table_hbm, idx_hbm, out_hbm, idx_v, rows_v, sem):
    wid = lax.axis_index("s") * NC + lax.axis_index("c")
    base = wid * (nchunk * CHUNK)
    pltpu.sync_copy(idx_hbm.at[wid], idx_v)

    def step(j, carry):
        pltpu.async_copy(table_hbm.at[idx_v.at[j]], rows_v, sem).wait()
        pltpu.sync_copy(rows_v, out_hbm.at[pl.ds(base + j * CHUNK, CHUNK)])
        return carry

    lax.fori_loop(0, nchunk, step, 0)


@jax.jit
def kernel(indices, table):
    b, h = indices.shape
    total = b * h
    nchunk = total // (NW * CHUNK)
    idx = indices.reshape(NW, nchunk, CHUNK).astype(jnp.int32)

    run = pl.kernel(
        functools.partial(_emb_body, nchunk),
        out_type=jax.ShapeDtypeStruct((total, DIM), jnp.float32),
        mesh=plsc.VectorSubcoreMesh(core_axis_name="c", subcore_axis_name="s"),
        scratch_types=[
            pltpu.VMEM((nchunk, CHUNK), jnp.int32),
            pltpu.VMEM((CHUNK, DIM), jnp.float32),
            pltpu.SemaphoreType.DMA,
        ],
    )
    out = run(table, idx)
    return out.reshape(b, h, DIM)


# speedup vs baseline: 3.3439x; 1.1236x over previous
import functools

import jax
import jax.numpy as jnp
from jax import lax
from jax.experimental import pallas as pl
from jax.experimental.pallas import tpu as pltpu
from jax.experimental.pallas import tpu_sc as plsc

DIM = 128
NC = 2
NS = 16
NW = NC * NS
CHUNK = 128
NBUF = 5


def _emb_body(nchunk, table_hbm, idx_hbm, out_hbm, idx_v, *bufs):
    rows = bufs[:NBUF]
    gsem = bufs[NBUF : 2 * NBUF]
    osem = bufs[2 * NBUF : 3 * NBUF]
    wid = lax.axis_index("s") * NC + lax.axis_index("c")
    base = wid * (nchunk * CHUNK)
    pltpu.sync_copy(idx_hbm.at[wid], idx_v)

    def gather_start(c, b):
        pltpu.async_copy(table_hbm.at[idx_v.at[c]], rows[b], gsem[b])

    def gather_wait(b):
        pltpu.make_async_copy(
            table_hbm.at[idx_v.at[0]], rows[b], gsem[b]
        ).wait()

    def out_wait(b):
        pltpu.make_async_copy(
            rows[b], out_hbm.at[pl.ds(0, CHUNK)], osem[b]
        ).wait()

    for b in range(NBUF):
        gather_start(b, b)

    nouter = nchunk // NBUF

    def outer(j2, carry):
        for b in range(NBUF):
            c = j2 * NBUF + b
            gather_wait(b)
            pltpu.async_copy(
                rows[b], out_hbm.at[pl.ds(base + c * CHUNK, CHUNK)], osem[b]
            )

            @pl.when(j2 < nouter - 1)
            def _():
                out_wait(b)
                gather_start(c + NBUF, b)

        return carry

    lax.fori_loop(0, nouter, outer, 0)
    for b in range(NBUF):
        out_wait(b)


@jax.jit
def kernel(indices, table):
    b, h = indices.shape
    total = b * h
    nchunk = total // (NW * CHUNK)
    idx = indices.reshape(NW, nchunk, CHUNK).astype(jnp.int32)

    run = pl.kernel(
        functools.partial(_emb_body, nchunk),
        out_type=jax.ShapeDtypeStruct((total, DIM), jnp.float32),
        mesh=plsc.VectorSubcoreMesh(core_axis_name="c", subcore_axis_name="s"),
        scratch_types=(
            [pltpu.VMEM((nchunk, CHUNK), jnp.int32)]
            + [pltpu.VMEM((CHUNK, DIM), jnp.float32) for _ in range(NBUF)]
            + [pltpu.SemaphoreType.DMA for _ in range(2 * NBUF)]
        ),
    )
    out = run(table, idx)
    return out.reshape(b, h, DIM)


# speedup vs baseline: 10.3172x; 3.0854x over previous
import functools

import jax
import jax.numpy as jnp
from jax import lax
from jax.experimental import pallas as pl
from jax.experimental.pallas import tpu as pltpu
from jax.experimental.pallas import tpu_sc as plsc

DIM = 128
NC = 2
NS = 16
NW = NC * NS
CHUNK = 256
NBUF = 3


def _emb_body(nchunk, table_hbm, idx_hbm, out_hbm, idx_v, *bufs):
    rows = bufs[:NBUF]
    gsem = bufs[NBUF : 2 * NBUF]
    osem = bufs[2 * NBUF : 3 * NBUF]
    wid = lax.axis_index("s") * NC + lax.axis_index("c")
    base = wid * (nchunk * CHUNK)
    pltpu.sync_copy(idx_hbm.at[pl.ds(base, nchunk * CHUNK)], idx_v)

    def gather_start(c, b):
        pltpu.async_copy(
            table_hbm.at[idx_v.at[pl.ds(c * CHUNK, CHUNK)]], rows[b], gsem[b]
        )

    def gather_wait(b):
        pltpu.make_async_copy(
            table_hbm.at[idx_v.at[pl.ds(0, CHUNK)]], rows[b], gsem[b]
        ).wait()

    def out_start(c, b):
        pltpu.async_copy(
            rows[b], out_hbm.at[pl.ds(base + c * CHUNK, CHUNK)], osem[b]
        )

    def out_wait(b):
        pltpu.make_async_copy(rows[b], out_hbm.at[pl.ds(0, CHUNK)], osem[b]).wait()

    nouter = nchunk // NBUF
    rem = nchunk - nouter * NBUF

    for b in range(NBUF):
        gather_start(b, b)

    def outer(j2, carry):
        for b in range(NBUF):
            c = j2 * NBUF + b
            gather_wait(b)
            out_start(c, b)

            @pl.when(j2 < nouter - 1)
            def _():
                out_wait(b)
                gather_start(c + NBUF, b)

        return carry

    lax.fori_loop(0, nouter, outer, 0)

    for r in range(rem):
        out_wait(r)
        gather_start(nouter * NBUF + r, r)
    for r in range(rem):
        gather_wait(r)
        out_start(nouter * NBUF + r, r)
    for b in range(NBUF):
        out_wait(b)


@jax.jit
def kernel(indices, table):
    b, h = indices.shape
    total = b * h
    nchunk = total // (NW * CHUNK)
    idx = indices.T.reshape(-1).astype(jnp.int32)

    run = pl.kernel(
        functools.partial(_emb_body, nchunk),
        out_type=jax.ShapeDtypeStruct((total, DIM), jnp.float32),
        mesh=plsc.VectorSubcoreMesh(core_axis_name="c", subcore_axis_name="s"),
        scratch_types=(
            [pltpu.VMEM((nchunk * CHUNK,), jnp.int32)]
            + [pltpu.VMEM((CHUNK, DIM), jnp.float32) for _ in range(NBUF)]
            + [pltpu.SemaphoreType.DMA for _ in range(2 * NBUF)]
        ),
    )
    out = run(table, idx)
    return out.reshape(h, b, DIM).transpose(1, 0, 2)


# speedup vs baseline: 10.4115x; 1.0091x over previous
import functools

import jax
import jax.numpy as jnp
from jax import lax
from jax.experimental import pallas as pl
from jax.experimental.pallas import tpu as pltpu
from jax.experimental.pallas import tpu_sc as plsc

DIM = 128
NC = 2
NS = 16
NW = NC * NS
CHUNK = 128
NBUF = 4


def _emb_body(nchunk, table_hbm, idx_hbm, out_hbm, idx_v, *bufs):
    rows = bufs[:NBUF]
    gsem = bufs[NBUF : 2 * NBUF]
    osem = bufs[2 * NBUF : 3 * NBUF]
    wid = lax.axis_index("s") * NC + lax.axis_index("c")
    base = wid * (nchunk * CHUNK)
    pltpu.sync_copy(idx_hbm.at[pl.ds(base, nchunk * CHUNK)], idx_v)

    def gather_start(c, b):
        pltpu.async_copy(
            table_hbm.at[idx_v.at[pl.ds(c * CHUNK, CHUNK)]], rows[b], gsem[b]
        )

    def gather_wait(b):
        pltpu.make_async_copy(
            table_hbm.at[idx_v.at[pl.ds(0, CHUNK)]], rows[b], gsem[b]
        ).wait()

    def out_start(c, b):
        pltpu.async_copy(
            rows[b], out_hbm.at[pl.ds(base + c * CHUNK, CHUNK)], osem[b]
        )

    def out_wait(b):
        pltpu.make_async_copy(rows[b], out_hbm.at[pl.ds(0, CHUNK)], osem[b]).wait()

    nouter = nchunk // NBUF
    rem = nchunk - nouter * NBUF

    for b in range(NBUF):
        gather_start(b, b)

    def outer(j2, carry):
        for b in range(NBUF):
            c = j2 * NBUF + b
            bp = (b - 1) % NBUF
            if b == 0:
                @pl.when(j2 > 0)
                def _():
                    out_wait(bp)
                    gather_start(c + NBUF - 1, bp)
            else:
                @pl.when(c + NBUF - 1 < nchunk)
                def _():
                    out_wait(bp)
                    gather_start(c + NBUF - 1, bp)

            gather_wait(b)
            out_start(c, b)

        return carry

    lax.fori_loop(0, nouter, outer, 0)

    for r in range(rem):
        c = nouter * NBUF + r
        b = c % NBUF
        gather_wait(b)
        out_start(c, b)
    for b in range(NBUF):
        out_wait(b)


@jax.jit
def kernel(indices, table):
    b, h = indices.shape
    total = b * h
    nchunk = total // (NW * CHUNK)
    idx = indices.T.reshape(-1).astype(jnp.int32)

    run = pl.kernel(
        functools.partial(_emb_body, nchunk),
        out_type=jax.ShapeDtypeStruct((total, DIM), jnp.float32),
        mesh=plsc.VectorSubcoreMesh(core_axis_name="c", subcore_axis_name="s"),
        scratch_types=(
            [pltpu.VMEM((nchunk * CHUNK,), jnp.int32)]
            + [pltpu.VMEM((CHUNK, DIM), jnp.float32) for _ in range(NBUF)]
            + [pltpu.SemaphoreType.DMA for _ in range(2 * NBUF)]
        ),
    )
    out = run(table, idx)
    return out.reshape(h, b, DIM).transpose(1, 0, 2)


# speedup vs baseline: 10.4206x; 1.0009x over previous
import functools

import jax
import jax.numpy as jnp
from jax import lax
from jax.experimental import pallas as pl
from jax.experimental.pallas import tpu as pltpu
from jax.experimental.pallas import tpu_sc as plsc

DIM = 128
NC = 2
NS = 16
NW = NC * NS
CHUNK = 128
NBUF = 5


def _emb_body(nchunk, table_hbm, idx_hbm, out_hbm, idx_v, *bufs):
    rows = bufs[:NBUF]
    gsem = bufs[NBUF : 2 * NBUF]
    osem = bufs[2 * NBUF : 3 * NBUF]
    wid = lax.axis_index("s") * NC + lax.axis_index("c")
    base = wid * (nchunk * CHUNK)
    pltpu.sync_copy(idx_hbm.at[wid], idx_v)

    def gather_start(c, b):
        pltpu.async_copy(table_hbm.at[idx_v.at[c]], rows[b], gsem[b])

    def gather_wait(b):
        pltpu.make_async_copy(
            table_hbm.at[idx_v.at[0]], rows[b], gsem[b]
        ).wait()

    def out_wait(b):
        pltpu.make_async_copy(
            rows[b], out_hbm.at[pl.ds(0, CHUNK)], osem[b]
        ).wait()

    for b in range(NBUF):
        gather_start(b, b)

    nouter = nchunk // NBUF

    def outer(j2, carry):
        for b in range(NBUF):
            c = j2 * NBUF + b
            gather_wait(b)
            pltpu.async_copy(
                rows[b], out_hbm.at[pl.ds(base + c * CHUNK, CHUNK)], osem[b]
            )

            @pl.when(j2 < nouter - 1)
            def _():
                out_wait(b)
                gather_start(c + NBUF, b)

        return carry

    lax.fori_loop(0, nouter, outer, 0)
    for b in range(NBUF):
        out_wait(b)


@jax.jit
def kernel(indices, table):
    b, h = indices.shape
    total = b * h
    nchunk = total // (NW * CHUNK)
    idx = indices.T.reshape(NW, nchunk, CHUNK).astype(jnp.int32)

    run = pl.kernel(
        functools.partial(_emb_body, nchunk),
        out_type=jax.ShapeDtypeStruct((total, DIM), jnp.float32),
        mesh=plsc.VectorSubcoreMesh(core_axis_name="c", subcore_axis_name="s"),
        scratch_types=(
            [pltpu.VMEM((nchunk, CHUNK), jnp.int32)]
            + [pltpu.VMEM((CHUNK, DIM), jnp.float32) for _ in range(NBUF)]
            + [pltpu.SemaphoreType.DMA for _ in range(2 * NBUF)]
        ),
    )
    out = run(table, idx)
    return out.reshape(h, b, DIM).transpose(1, 0, 2)
